# baseline (speedup 1.0000x reference)
import jax
import jax.numpy as jnp
from jax.experimental import pallas as pl

_BM = 256


def _s1_body(x_ref, w1_ref, o_ref):
    o_ref[...] = jnp.dot(x_ref[...], w1_ref[...],
                         preferred_element_type=jnp.float32)


def _s2_body(adj_ref, s1_ref, w2_ref, o_ref):
    h = jnp.maximum(
        jnp.dot(adj_ref[...], s1_ref[...], preferred_element_type=jnp.float32),
        0.0)
    o_ref[...] = jnp.dot(h, w2_ref[...], preferred_element_type=jnp.float32)


def _mu_body(adj_ref, s2_ref, o_ref):
    o_ref[...] = jnp.dot(adj_ref[...], s2_ref[...],
                         preferred_element_type=jnp.float32)


def _recon_body(mu_i_ref, mu_t_ref, o_ref):
    o_ref[...] = jnp.dot(mu_i_ref[...], mu_t_ref[...],
                         preferred_element_type=jnp.float32)


def kernel(x, adj, W1, W2):
    n, nfeat = x.shape
    nhid = W1.shape[1]
    ncls = W2.shape[1]
    grid_m = pl.cdiv(n, _BM)

    s1 = pl.pallas_call(
        _s1_body,
        out_shape=jax.ShapeDtypeStruct((n, nhid), jnp.float32),
    )(x, W1)

    s2 = pl.pallas_call(
        _s2_body,
        grid=(grid_m,),
        in_specs=[
            pl.BlockSpec((_BM, n), lambda i: (i, 0)),
            pl.BlockSpec((n, nhid), lambda i: (0, 0)),
            pl.BlockSpec((nhid, ncls), lambda i: (0, 0)),
        ],
        out_specs=pl.BlockSpec((_BM, ncls), lambda i: (i, 0)),
        out_shape=jax.ShapeDtypeStruct((n, ncls), jnp.float32),
    )(adj, s1, W2)

    mu = pl.pallas_call(
        _mu_body,
        grid=(grid_m,),
        in_specs=[
            pl.BlockSpec((_BM, n), lambda i: (i, 0)),
            pl.BlockSpec((n, ncls), lambda i: (0, 0)),
        ],
        out_specs=pl.BlockSpec((_BM, ncls), lambda i: (i, 0)),
        out_shape=jax.ShapeDtypeStruct((n, ncls), jnp.float32),
    )(adj, s2)

    mu_t = mu.T

    recon = pl.pallas_call(
        _recon_body,
        grid=(grid_m,),
        in_specs=[
            pl.BlockSpec((_BM, ncls), lambda i: (i, 0)),
            pl.BlockSpec((ncls, n), lambda i: (0, 0)),
        ],
        out_specs=pl.BlockSpec((_BM, n), lambda i: (i, 0)),
        out_shape=jax.ShapeDtypeStruct((n, n), jnp.float32),
    )(mu, mu_t)

    return recon


# speedup vs baseline: 1.0341x; 1.0341x over previous
import jax
import jax.numpy as jnp
from jax.experimental import pallas as pl
from jax.experimental.pallas import tpu as pltpu

_BM = 200


def _fused_body(x_ref, adj_ref, w1_ref, w2_ref, o_ref,
                s1_ref, s2_ref, mu_ref, mut_ref, *, gm, bm):
    s = pl.program_id(0)

    @pl.when(s == 0)
    def _():
        s1_ref[...] = jnp.dot(x_ref[...], w1_ref[...],
                              preferred_element_type=jnp.float32)

    @pl.when(s < gm)
    def _():
        h = jnp.maximum(
            jnp.dot(adj_ref[...], s1_ref[...],
                    preferred_element_type=jnp.float32), 0.0)
        s2_ref[pl.ds(s * bm, bm), :] = jnp.dot(
            h, w2_ref[...], preferred_element_type=jnp.float32)

    @pl.when((s >= gm) & (s < 2 * gm))
    def _():
        mu_ref[pl.ds((s - gm) * bm, bm), :] = jnp.dot(
            adj_ref[...], s2_ref[...], preferred_element_type=jnp.float32)

    @pl.when(s == 2 * gm)
    def _():
        mut_ref[...] = mu_ref[...].T

    @pl.when(s >= 2 * gm)
    def _():
        k = s - 2 * gm
        o_ref[...] = jnp.dot(mu_ref[pl.ds(k * bm, bm), :], mut_ref[...],
                             preferred_element_type=jnp.float32)


def kernel(x, adj, W1, W2):
    n, nfeat = x.shape
    nhid = W1.shape[1]
    ncls = W2.shape[1]
    gm = n // _BM
    assert gm * _BM == n

    import functools
    body = functools.partial(_fused_body, gm=gm, bm=_BM)

    return pl.pallas_call(
        body,
        grid=(3 * gm,),
        in_specs=[
            pl.BlockSpec((n, nfeat), lambda s: (0, 0)),
            pl.BlockSpec((_BM, n),
                         lambda s: (jnp.where(s < 2 * gm, s % gm, gm - 1), 0)),
            pl.BlockSpec((nfeat, nhid), lambda s: (0, 0)),
            pl.BlockSpec((nhid, ncls), lambda s: (0, 0)),
        ],
        out_specs=pl.BlockSpec(
            (_BM, n), lambda s: (jnp.where(s < 2 * gm, 0, s - 2 * gm), 0)),
        out_shape=jax.ShapeDtypeStruct((n, n), jnp.float32),
        scratch_shapes=[
            pltpu.VMEM((n, nhid), jnp.float32),
            pltpu.VMEM((n, ncls), jnp.float32),
            pltpu.VMEM((n, ncls), jnp.float32),
            pltpu.VMEM((ncls, n), jnp.float32),
        ],
    )(x, adj, W1, W2)


# speedup vs baseline: 1.0348x; 1.0007x over previous
import functools

import jax
import jax.numpy as jnp
from jax.experimental import pallas as pl
from jax.experimental.pallas import tpu as pltpu

_BM = 400


def _ab_body(x_ref, adj_ref, w1_ref, w2_ref, mu_ref,
             s1_ref, s2_ref, *, gm, bm):
    s = pl.program_id(0)

    @pl.when(s == 0)
    def _():
        s1_ref[...] = jnp.dot(x_ref[...], w1_ref[...],
                              preferred_element_type=jnp.float32)

    @pl.when(s < gm)
    def _():
        h = jnp.maximum(
            jnp.dot(adj_ref[...], s1_ref[...],
                    preferred_element_type=jnp.float32), 0.0)
        s2_ref[pl.ds(s * bm, bm), :] = jnp.dot(
            h, w2_ref[...], preferred_element_type=jnp.float32)

    @pl.when(s >= gm)
    def _():
        mu_ref[...] = jnp.dot(adj_ref[...], s2_ref[...],
                              preferred_element_type=jnp.float32)


def _c_body(mu_ref, o_ref, mut_ref, *, bm):
    s = pl.program_id(0)

    @pl.when(s == 0)
    def _():
        mut_ref[...] = mu_ref[...].T

    o_ref[...] = jnp.dot(mu_ref[pl.ds(s * bm, bm), :], mut_ref[...],
                         preferred_element_type=jnp.float32)


def kernel(x, adj, W1, W2):
    n, nfeat = x.shape
    nhid = W1.shape[1]
    ncls = W2.shape[1]
    gm = n // _BM
    assert gm * _BM == n

    mu = pl.pallas_call(
        functools.partial(_ab_body, gm=gm, bm=_BM),
        grid=(2 * gm,),
        in_specs=[
            pl.BlockSpec((n, nfeat), lambda s: (0, 0)),
            pl.BlockSpec((_BM, n), lambda s: (s % gm, 0)),
            pl.BlockSpec((nfeat, nhid), lambda s: (0, 0)),
            pl.BlockSpec((nhid, ncls), lambda s: (0, 0)),
        ],
        out_specs=pl.BlockSpec(
            (_BM, ncls), lambda s: (jnp.where(s < gm, 0, s - gm), 0)),
        out_shape=jax.ShapeDtypeStruct((n, ncls), jnp.float32),
        scratch_shapes=[
            pltpu.VMEM((n, nhid), jnp.float32),
            pltpu.VMEM((n, ncls), jnp.float32),
        ],
    )(x, adj, W1, W2)

    recon = pl.pallas_call(
        functools.partial(_c_body, bm=_BM),
        grid=(gm,),
        in_specs=[pl.BlockSpec((n, ncls), lambda s: (0, 0))],
        out_specs=pl.BlockSpec((_BM, n), lambda s: (s, 0)),
        out_shape=jax.ShapeDtypeStruct((n, n), jnp.float32),
        scratch_shapes=[pltpu.VMEM((ncls, n), jnp.float32)],
    )(mu)

    return recon
